# baseline (speedup 1.0000x reference)
import functools

import jax
import jax.numpy as jnp
from jax import lax
from jax.experimental import pallas as pl
from jax.experimental.pallas import tpu as pltpu
from jax.experimental.pallas import tpu_sc as plsc

NCLS = 80
NPROP = 20000
NGT = 100
GPAD = 112
L = 16
GB = 8


def _body(nc, ns, pw, tl, px1, py1, px2, py2, g1, g2, g3, g4, gcls,
          ovals, oidxs, ocls, ob1, ob2, ob3, ob4, ooh,
          pv1, pv2, pv3, pv4, vg, vcls, vb, vmi,
          sv, si, sc, sb1, sb2, sb3, sb4, voh, sem, psem):
    wid = lax.axis_index("s") * nc + lax.axis_index("c")
    nw = nc * ns
    base = wid * pw
    last = wid == nw - 1

    cps = [
        pltpu.async_copy(g1, vg.at[pl.ds(0, NGT)], sem),
        pltpu.async_copy(g2, vg.at[pl.ds(GPAD, NGT)], sem),
        pltpu.async_copy(g3, vg.at[pl.ds(2 * GPAD, NGT)], sem),
        pltpu.async_copy(g4, vg.at[pl.ds(3 * GPAD, NGT)], sem),
        pltpu.async_copy(gcls, vcls.at[pl.ds(0, NGT)], sem),
    ]

    @pl.when(jnp.logical_not(last))
    def _():
        pltpu.async_copy(px1.at[pl.ds(base, pw)], pv1, psem)
        pltpu.async_copy(py1.at[pl.ds(base, pw)], pv2, psem)
        pltpu.async_copy(px2.at[pl.ds(base, pw)], pv3, psem)
        pltpu.async_copy(py2.at[pl.ds(base, pw)], pv4, psem)

    @pl.when(last)
    def _():
        ds = pl.ds(base, tl)
        dd = pl.ds(0, tl)
        pltpu.async_copy(px1.at[ds], pv1.at[dd], psem)
        pltpu.async_copy(py1.at[ds], pv2.at[dd], psem)
        pltpu.async_copy(px2.at[ds], pv3.at[dd], psem)
        pltpu.async_copy(py2.at[ds], pv4.at[dd], psem)

    for c in cps:
        c.wait()

    lane = lax.iota(jnp.int32, L)

    def bcast(m, _):
        sidx = jnp.full((L,), m, jnp.int32)
        gx1 = plsc.load_gather(vg, [sidx])
        gy1 = plsc.load_gather(vg, [sidx + GPAD])
        gx2 = plsc.load_gather(vg, [sidx + 2 * GPAD])
        gy2 = plsc.load_gather(vg, [sidx + 3 * GPAD])
        mo = m * L
        vb[pl.ds(mo, L)] = gx1
        vb[pl.ds(NGT * L + mo, L)] = gy1
        vb[pl.ds(2 * NGT * L + mo, L)] = gx2
        vb[pl.ds(3 * NGT * L + mo, L)] = gy2
        vb[pl.ds(4 * NGT * L + mo, L)] = (gx2 - gx1) * (gy2 - gy1)
        vmi[pl.ds(mo, L)] = sidx
        return 0

    lax.fori_loop(0, NGT, bcast, 0)

    @pl.when(jnp.logical_not(last))
    def _():
        pltpu.make_async_copy(px1.at[pl.ds(base, pw)], pv1, psem).wait()
        pltpu.make_async_copy(py1.at[pl.ds(base, pw)], pv2, psem).wait()
        pltpu.make_async_copy(px2.at[pl.ds(base, pw)], pv3, psem).wait()
        pltpu.make_async_copy(py2.at[pl.ds(base, pw)], pv4, psem).wait()

    @pl.when(last)
    def _():
        ds = pl.ds(base, tl)
        dd = pl.ds(0, tl)
        pltpu.make_async_copy(px1.at[ds], pv1.at[dd], psem).wait()
        pltpu.make_async_copy(py1.at[ds], pv2.at[dd], psem).wait()
        pltpu.make_async_copy(px2.at[ds], pv3.at[dd], psem).wait()
        pltpu.make_async_copy(py2.at[ds], pv4.at[dd], psem).wait()

    nblk = pw // (GB * L)

    @plsc.parallel_loop(0, nblk, step=1)
    def bbody(b):
        off = b * (GB * L)
        offs = [off + j * L for j in range(GB)]
        x1s = [pv1[pl.ds(o, L)] for o in offs]
        y1s = [pv2[pl.ds(o, L)] for o in offs]
        x2s = [pv3[pl.ds(o, L)] for o in offs]
        y2s = [pv4[pl.ds(o, L)] for o in offs]
        pas = [(x2s[j] - x1s[j]) * (y2s[j] - y1s[j]) for j in range(GB)]

        zf = jnp.zeros((L,), jnp.float32)
        onef = jnp.ones((L,), jnp.float32)
        zi = jnp.zeros((L,), jnp.int32)
        init = (tuple(zf for _ in range(GB)),
                tuple(onef for _ in range(GB)),
                tuple(zi for _ in range(GB)))

        def mbody(m, carry):
            bis, bus, bids = carry
            mo = m * L
            gx1 = vb[pl.ds(mo, L)]
            gy1 = vb[pl.ds(NGT * L + mo, L)]
            gx2 = vb[pl.ds(2 * NGT * L + mo, L)]
            gy2 = vb[pl.ds(3 * NGT * L + mo, L)]
            ga = vb[pl.ds(4 * NGT * L + mo, L)]
            midx = vmi[pl.ds(mo, L)]
            nbi, nbu, nbd = [], [], []
            for j in range(GB):
                ltx = jnp.maximum(gx1, x1s[j])
                lty = jnp.maximum(gy1, y1s[j])
                rbx = jnp.minimum(gx2, x2s[j])
                rby = jnp.minimum(gy2, y2s[j])
                w = jnp.maximum(rbx - ltx, 0.0)
                inter = w * (rby - lty)
                sums = ga + pas[j]
                upd = inter * bus[j] > bis[j] * sums
                nbi.append(jnp.where(upd, inter, bis[j]))
                nbu.append(jnp.where(upd, sums, bus[j]))
                nbd.append(jnp.where(upd, midx, bids[j]))
            return (tuple(nbi), tuple(nbu), tuple(nbd))

        bis, bus, bids = lax.fori_loop(0, NGT, mbody, init)

        for j in range(GB):
            o = offs[j]
            vals = bis[j] / (bus[j] - bis[j])
            fg = vals >= 0.5
            idx = bids[j]
            cls = plsc.load_gather(vcls, [idx])
            cls = jnp.where(fg, cls, NCLS)
            sv[pl.ds(o, L)] = vals
            si[pl.ds(o, L)] = idx
            sc[pl.ds(o, L)] = cls
            sb1[pl.ds(o, L)] = plsc.load_gather(vg, [idx])
            sb2[pl.ds(o, L)] = plsc.load_gather(vg, [idx + GPAD])
            sb3[pl.ds(o, L)] = plsc.load_gather(vg, [idx + 2 * GPAD])
            sb4[pl.ds(o, L)] = plsc.load_gather(vg, [idx + 3 * GPAD])

    @pl.when(wid == 0)
    def _():
        zf16 = jnp.zeros((L,), jnp.float32)
        for c in range(96 // L):
            voh[pl.ds(c * L, L)] = zf16
        voh[pl.ds(NCLS, L)] = jnp.where(lane == 0, 1.0, 0.0)
        ones = jnp.ones((L,), jnp.float32)
        for c in range(GPAD // L):
            ids = vcls[pl.ds(c * L, L)]
            if (c + 1) * L <= NGT:
                plsc.store_scatter(voh, [ids], ones)
            else:
                plsc.store_scatter(voh, [ids], ones,
                                   mask=lane + c * L < NGT)
        pltpu.sync_copy(voh.at[pl.ds(0, NCLS + 1)], ooh)

    @pl.when(jnp.logical_not(last))
    def _():
        sl = pl.ds(base, pw)
        for c in [pltpu.async_copy(sv, ovals.at[sl], sem),
                  pltpu.async_copy(si, oidxs.at[sl], sem),
                  pltpu.async_copy(sc, ocls.at[sl], sem),
                  pltpu.async_copy(sb1, ob1.at[sl], sem),
                  pltpu.async_copy(sb2, ob2.at[sl], sem),
                  pltpu.async_copy(sb3, ob3.at[sl], sem),
                  pltpu.async_copy(sb4, ob4.at[sl], sem)]:
            c.wait()

    @pl.when(last)
    def _():
        sl = pl.ds(base, tl)
        dd = pl.ds(0, tl)
        for c in [pltpu.async_copy(sv.at[dd], ovals.at[sl], sem),
                  pltpu.async_copy(si.at[dd], oidxs.at[sl], sem),
                  pltpu.async_copy(sc.at[dd], ocls.at[sl], sem),
                  pltpu.async_copy(sb1.at[dd], ob1.at[sl], sem),
                  pltpu.async_copy(sb2.at[dd], ob2.at[sl], sem),
                  pltpu.async_copy(sb3.at[dd], ob3.at[sl], sem),
                  pltpu.async_copy(sb4.at[dd], ob4.at[sl], sem)]:
            c.wait()


def kernel(proposal_boxes, gt_boxes, gt_classes):
    try:
        info = plsc.get_sparse_core_info()
        nc, ns = info.num_cores, info.num_subcores
    except Exception:
        nc, ns = 2, 16
    nw = nc * ns
    blk = GB * L
    pw = (-(-NPROP // nw) + blk - 1) // blk * blk
    tl = NPROP - (nw - 1) * pw

    px1, py1, px2, py2 = (proposal_boxes[:, k] for k in range(4))
    g1, g2, g3, g4 = (gt_boxes[:, k] for k in range(4))
    gcls = gt_classes.astype(jnp.int32)

    mesh = plsc.VectorSubcoreMesh(core_axis_name="c", subcore_axis_name="s",
                                  num_cores=nc, num_subcores=ns)
    f32, i32 = jnp.float32, jnp.int32
    out_type = (
        jax.ShapeDtypeStruct((NPROP,), f32),
        jax.ShapeDtypeStruct((NPROP,), i32),
        jax.ShapeDtypeStruct((NPROP,), i32),
        jax.ShapeDtypeStruct((NPROP,), f32),
        jax.ShapeDtypeStruct((NPROP,), f32),
        jax.ShapeDtypeStruct((NPROP,), f32),
        jax.ShapeDtypeStruct((NPROP,), f32),
        jax.ShapeDtypeStruct((NCLS + 1,), f32),
    )
    scratch = [
        pltpu.VMEM((pw,), f32), pltpu.VMEM((pw,), f32),
        pltpu.VMEM((pw,), f32), pltpu.VMEM((pw,), f32),
        pltpu.VMEM((4 * GPAD,), f32),
        pltpu.VMEM((GPAD,), i32),
        pltpu.VMEM((5 * NGT * L,), f32),
        pltpu.VMEM((NGT * L,), i32),
        pltpu.VMEM((pw,), f32), pltpu.VMEM((pw,), i32),
        pltpu.VMEM((pw,), i32),
        pltpu.VMEM((pw,), f32), pltpu.VMEM((pw,), f32),
        pltpu.VMEM((pw,), f32), pltpu.VMEM((pw,), f32),
        pltpu.VMEM((96,), f32),
        pltpu.SemaphoreType.DMA,
        pltpu.SemaphoreType.DMA,
    ]
    run = pl.kernel(functools.partial(_body, nc, ns, pw, tl),
                    out_type=out_type, mesh=mesh, scratch_types=scratch,
                    compiler_params=pltpu.CompilerParams(
                        needs_layout_passes=False))
    vals, idxs, cls, b1, b2, b3, b4, oh = run(
        px1, py1, px2, py2, g1, g2, g3, g4, gcls)
    boxes = jnp.stack([b1, b2, b3, b4], axis=1)
    return (vals, idxs, cls, boxes, oh)


# speedup vs baseline: 1.5941x; 1.5941x over previous
import functools

import jax
import jax.numpy as jnp
from jax import lax
from jax.experimental import pallas as pl
from jax.experimental.pallas import tpu as pltpu
from jax.experimental.pallas import tpu_sc as plsc

NCLS = 80
NPROP = 20000
NGT = 100
GPAD = 112
L = 16
GB = 5


def _body(nc, ns, pw, tl, px1, py1, px2, py2, g1, g2, g3, g4, gcls,
          ovals, oidxs, ocls, ob1, ob2, ob3, ob4, ooh,
          pv1, pv2, pv3, pv4, vg, vcls, vb, vmi,
          sv, si, sc, sb1, sb2, sb3, sb4, voh, sem, psem):
    wid = lax.axis_index("s") * nc + lax.axis_index("c")
    nw = nc * ns
    base = wid * pw
    last = wid == nw - 1

    cps = [
        pltpu.async_copy(g1, vg.at[pl.ds(0, NGT)], sem),
        pltpu.async_copy(g2, vg.at[pl.ds(GPAD, NGT)], sem),
        pltpu.async_copy(g3, vg.at[pl.ds(2 * GPAD, NGT)], sem),
        pltpu.async_copy(g4, vg.at[pl.ds(3 * GPAD, NGT)], sem),
        pltpu.async_copy(gcls, vcls.at[pl.ds(0, NGT)], sem),
    ]

    @pl.when(jnp.logical_not(last))
    def _():
        pltpu.async_copy(px1.at[pl.ds(base, pw)], pv1, psem)
        pltpu.async_copy(py1.at[pl.ds(base, pw)], pv2, psem)
        pltpu.async_copy(px2.at[pl.ds(base, pw)], pv3, psem)
        pltpu.async_copy(py2.at[pl.ds(base, pw)], pv4, psem)

    @pl.when(last)
    def _():
        ds = pl.ds(base, tl)
        dd = pl.ds(0, tl)
        pltpu.async_copy(px1.at[ds], pv1.at[dd], psem)
        pltpu.async_copy(py1.at[ds], pv2.at[dd], psem)
        pltpu.async_copy(px2.at[ds], pv3.at[dd], psem)
        pltpu.async_copy(py2.at[ds], pv4.at[dd], psem)

    for c in cps:
        c.wait()

    lane = lax.iota(jnp.int32, L)

    def bcast(m, _):
        sidx = jnp.full((L,), m, jnp.int32)
        gx1 = plsc.load_gather(vg, [sidx])
        gy1 = plsc.load_gather(vg, [sidx + GPAD])
        gx2 = plsc.load_gather(vg, [sidx + 2 * GPAD])
        gy2 = plsc.load_gather(vg, [sidx + 3 * GPAD])
        mo = m * L
        vb[pl.ds(mo, L)] = gx1
        vb[pl.ds(NGT * L + mo, L)] = gy1
        vb[pl.ds(2 * NGT * L + mo, L)] = gx2
        vb[pl.ds(3 * NGT * L + mo, L)] = gy2
        vb[pl.ds(4 * NGT * L + mo, L)] = (gx2 - gx1) * (gy2 - gy1)
        vmi[pl.ds(mo, L)] = sidx
        return 0

    lax.fori_loop(0, NGT, bcast, 0)

    @pl.when(jnp.logical_not(last))
    def _():
        pltpu.make_async_copy(px1.at[pl.ds(base, pw)], pv1, psem).wait()
        pltpu.make_async_copy(py1.at[pl.ds(base, pw)], pv2, psem).wait()
        pltpu.make_async_copy(px2.at[pl.ds(base, pw)], pv3, psem).wait()
        pltpu.make_async_copy(py2.at[pl.ds(base, pw)], pv4, psem).wait()

    @pl.when(last)
    def _():
        ds = pl.ds(base, tl)
        dd = pl.ds(0, tl)
        pltpu.make_async_copy(px1.at[ds], pv1.at[dd], psem).wait()
        pltpu.make_async_copy(py1.at[ds], pv2.at[dd], psem).wait()
        pltpu.make_async_copy(px2.at[ds], pv3.at[dd], psem).wait()
        pltpu.make_async_copy(py2.at[ds], pv4.at[dd], psem).wait()

    nblk = pw // (GB * L)

    @plsc.parallel_loop(0, nblk, step=1)
    def bbody(b):
        off = b * (GB * L)
        offs = [off + j * L for j in range(GB)]
        x1s = [pv1[pl.ds(o, L)] for o in offs]
        y1s = [pv2[pl.ds(o, L)] for o in offs]
        x2s = [pv3[pl.ds(o, L)] for o in offs]
        y2s = [pv4[pl.ds(o, L)] for o in offs]
        pas = [(x2s[j] - x1s[j]) * (y2s[j] - y1s[j]) for j in range(GB)]

        zf = jnp.zeros((L,), jnp.float32)
        onef = jnp.ones((L,), jnp.float32)
        zi = jnp.zeros((L,), jnp.int32)
        init = (tuple(zf for _ in range(GB)),
                tuple(onef for _ in range(GB)),
                tuple(zi for _ in range(GB)))

        def mbody(m, carry):
            bis, bus, bids = carry
            mo = m * L
            gx1 = vb[pl.ds(mo, L)]
            gy1 = vb[pl.ds(NGT * L + mo, L)]
            gx2 = vb[pl.ds(2 * NGT * L + mo, L)]
            gy2 = vb[pl.ds(3 * NGT * L + mo, L)]
            ga = vb[pl.ds(4 * NGT * L + mo, L)]
            midx = vmi[pl.ds(mo, L)]
            nbi, nbu, nbd = [], [], []
            for j in range(GB):
                ltx = jnp.maximum(gx1, x1s[j])
                lty = jnp.maximum(gy1, y1s[j])
                rbx = jnp.minimum(gx2, x2s[j])
                rby = jnp.minimum(gy2, y2s[j])
                w = jnp.maximum(rbx - ltx, 0.0)
                inter = w * (rby - lty)
                sums = ga + pas[j]
                upd = inter * bus[j] > bis[j] * sums
                nbi.append(jnp.where(upd, inter, bis[j]))
                nbu.append(jnp.where(upd, sums, bus[j]))
                nbd.append(jnp.where(upd, midx, bids[j]))
            return (tuple(nbi), tuple(nbu), tuple(nbd))

        bis, bus, bids = lax.fori_loop(0, NGT, mbody, init)

        for j in range(GB):
            o = offs[j]
            vals = bis[j] / (bus[j] - bis[j])
            fg = vals >= 0.5
            idx = bids[j]
            cls = plsc.load_gather(vcls, [idx])
            cls = jnp.where(fg, cls, NCLS)
            sv[pl.ds(o, L)] = vals
            si[pl.ds(o, L)] = idx
            sc[pl.ds(o, L)] = cls
            sb1[pl.ds(o, L)] = plsc.load_gather(vg, [idx])
            sb2[pl.ds(o, L)] = plsc.load_gather(vg, [idx + GPAD])
            sb3[pl.ds(o, L)] = plsc.load_gather(vg, [idx + 2 * GPAD])
            sb4[pl.ds(o, L)] = plsc.load_gather(vg, [idx + 3 * GPAD])

    @pl.when(wid == 0)
    def _():
        zf16 = jnp.zeros((L,), jnp.float32)
        for c in range(96 // L):
            voh[pl.ds(c * L, L)] = zf16
        voh[pl.ds(NCLS, L)] = jnp.where(lane == 0, 1.0, 0.0)
        ones = jnp.ones((L,), jnp.float32)
        for c in range(GPAD // L):
            ids = vcls[pl.ds(c * L, L)]
            if (c + 1) * L <= NGT:
                plsc.store_scatter(voh, [ids], ones)
            else:
                plsc.store_scatter(voh, [ids], ones,
                                   mask=lane + c * L < NGT)
        pltpu.sync_copy(voh.at[pl.ds(0, NCLS + 1)], ooh)

    @pl.when(jnp.logical_not(last))
    def _():
        sl = pl.ds(base, pw)
        for c in [pltpu.async_copy(sv, ovals.at[sl], sem),
                  pltpu.async_copy(si, oidxs.at[sl], sem),
                  pltpu.async_copy(sc, ocls.at[sl], sem),
                  pltpu.async_copy(sb1, ob1.at[sl], sem),
                  pltpu.async_copy(sb2, ob2.at[sl], sem),
                  pltpu.async_copy(sb3, ob3.at[sl], sem),
                  pltpu.async_copy(sb4, ob4.at[sl], sem)]:
            c.wait()

    @pl.when(last)
    def _():
        sl = pl.ds(base, tl)
        dd = pl.ds(0, tl)
        for c in [pltpu.async_copy(sv.at[dd], ovals.at[sl], sem),
                  pltpu.async_copy(si.at[dd], oidxs.at[sl], sem),
                  pltpu.async_copy(sc.at[dd], ocls.at[sl], sem),
                  pltpu.async_copy(sb1.at[dd], ob1.at[sl], sem),
                  pltpu.async_copy(sb2.at[dd], ob2.at[sl], sem),
                  pltpu.async_copy(sb3.at[dd], ob3.at[sl], sem),
                  pltpu.async_copy(sb4.at[dd], ob4.at[sl], sem)]:
            c.wait()


def kernel(proposal_boxes, gt_boxes, gt_classes):
    try:
        info = plsc.get_sparse_core_info()
        nc, ns = info.num_cores, info.num_subcores
    except Exception:
        nc, ns = 2, 16
    nw = nc * ns
    blk = GB * L
    pw = (-(-NPROP // nw) + blk - 1) // blk * blk
    tl = NPROP - (nw - 1) * pw

    px1, py1, px2, py2 = (proposal_boxes[:, k] for k in range(4))
    g1, g2, g3, g4 = (gt_boxes[:, k] for k in range(4))
    gcls = gt_classes.astype(jnp.int32)

    mesh = plsc.VectorSubcoreMesh(core_axis_name="c", subcore_axis_name="s",
                                  num_cores=nc, num_subcores=ns)
    f32, i32 = jnp.float32, jnp.int32
    out_type = (
        jax.ShapeDtypeStruct((NPROP,), f32),
        jax.ShapeDtypeStruct((NPROP,), i32),
        jax.ShapeDtypeStruct((NPROP,), i32),
        jax.ShapeDtypeStruct((NPROP,), f32),
        jax.ShapeDtypeStruct((NPROP,), f32),
        jax.ShapeDtypeStruct((NPROP,), f32),
        jax.ShapeDtypeStruct((NPROP,), f32),
        jax.ShapeDtypeStruct((NCLS + 1,), f32),
    )
    scratch = [
        pltpu.VMEM((pw,), f32), pltpu.VMEM((pw,), f32),
        pltpu.VMEM((pw,), f32), pltpu.VMEM((pw,), f32),
        pltpu.VMEM((4 * GPAD,), f32),
        pltpu.VMEM((GPAD,), i32),
        pltpu.VMEM((5 * NGT * L,), f32),
        pltpu.VMEM((NGT * L,), i32),
        pltpu.VMEM((pw,), f32), pltpu.VMEM((pw,), i32),
        pltpu.VMEM((pw,), i32),
        pltpu.VMEM((pw,), f32), pltpu.VMEM((pw,), f32),
        pltpu.VMEM((pw,), f32), pltpu.VMEM((pw,), f32),
        pltpu.VMEM((96,), f32),
        pltpu.SemaphoreType.DMA,
        pltpu.SemaphoreType.DMA,
    ]
    run = pl.kernel(functools.partial(_body, nc, ns, pw, tl),
                    out_type=out_type, mesh=mesh, scratch_types=scratch,
                    compiler_params=pltpu.CompilerParams(
                        needs_layout_passes=False))
    vals, idxs, cls, b1, b2, b3, b4, oh = run(
        px1, py1, px2, py2, g1, g2, g3, g4, gcls)
    boxes = jnp.stack([b1, b2, b3, b4], axis=1)
    return (vals, idxs, cls, boxes, oh)
